# baseline (speedup 1.0000x reference)
import functools

import jax
import jax.numpy as jnp
from jax import lax
from jax.experimental import pallas as pl
from jax.experimental.pallas import tpu as pltpu
from jax.experimental.pallas import tpu_sc as plsc


def _make_gather_sum(vocab_rows, emb, batch, ctx):
  info = plsc.get_sparse_core_info()
  nc, ns = info.num_cores, info.num_subcores
  nw = nc * ns
  bpw = batch // nw
  ipw = bpw * ctx
  chunk = 128
  nchunk = ipw // chunk
  assert ipw % chunk == 0

  mesh = plsc.VectorSubcoreMesh(core_axis_name="c", subcore_axis_name="s")

  @functools.partial(
      pl.kernel,
      mesh=mesh,
      out_type=jax.ShapeDtypeStruct((batch, emb), jnp.float32),
      compiler_params=pltpu.CompilerParams(use_tc_tiling_on_sc=False),
      scratch_types=[
          pltpu.VMEM((nchunk, chunk), jnp.int32),
          pltpu.VMEM((ipw, emb), jnp.float32),
          pltpu.VMEM((bpw, emb), jnp.float32),
          pltpu.SemaphoreType.DMA,
      ],
  )
  def gather_sum(idx_hbm, table_hbm, out_hbm, idx_v, rows_v, acc_v, sem):
    wid = lax.axis_index("s") * nc + lax.axis_index("c")
    pltpu.sync_copy(idx_hbm.at[wid], idx_v)
    copies = []
    for j in range(nchunk):
      copies.append(
          pltpu.async_copy(
              table_hbm.at[idx_v.at[j]],
              rows_v.at[pl.ds(j * chunk, chunk)],
              sem,
          ))
    for c in copies:
      c.wait()

    def body(i, carry):
      for jj in range(emb // 16):
        sl = pl.ds(jj * 16, 16)
        acc = rows_v[i * ctx, sl]
        for c in range(1, ctx):
          acc = acc + rows_v[i * ctx + c, sl]
        acc_v[i, sl] = acc
      return carry

    lax.fori_loop(0, bpw, body, 0)
    pltpu.sync_copy(acc_v, out_hbm.at[pl.ds(wid * bpw, bpw)])

  return gather_sum


_TV = 1024
_LOG2E = 1.4426950408889634


def _stats_body(nv, emb_ref, w_ref, b_ref, logz_ref, m_ref, s_ref):
  v = pl.program_id(0)

  @pl.when(v == 0)
  def _():
    m_ref[...] = jnp.full_like(m_ref, -jnp.inf)
    s_ref[...] = jnp.zeros_like(s_ref)

  logits2 = lax.dot_general(
      emb_ref[...], w_ref[...],
      dimension_numbers=(((1,), (1,)), ((), ())),
      preferred_element_type=jnp.float32,
  ) + b_ref[...]
  tile_max = jnp.max(logits2, axis=1, keepdims=True)
  m_old = m_ref[...]
  m_new = jnp.maximum(m_old, tile_max)
  s_ref[...] = (s_ref[...] * jnp.exp2(m_old - m_new)
                + jnp.sum(jnp.exp2(logits2 - m_new), axis=1, keepdims=True))
  m_ref[...] = m_new

  @pl.when(v == nv - 1)
  def _():
    logz_ref[...] = (m_ref[...] + jnp.log2(s_ref[...])) * (1.0 / _LOG2E)


def _project_body(emb_ref, w_ref, b_ref, logz_ref, out_ref):
  acc = lax.dot_general(
      emb_ref[...], w_ref[...],
      dimension_numbers=(((1,), (1,)), ((), ())),
      preferred_element_type=jnp.float32,
  )
  out_ref[...] = acc + (b_ref[...] - logz_ref[...])


def _log_softmax_linear(embeds, W, b):
  batch, emb = embeds.shape
  vocab = W.shape[0]
  nv = pl.cdiv(vocab, _TV)
  vpad = nv * _TV - vocab

  w16 = jnp.pad(W, ((0, vpad), (0, 0))).astype(jnp.bfloat16)
  b2s = jnp.pad(b * _LOG2E, (0, vpad), constant_values=-1e30).reshape(1, -1)
  emb16 = (embeds * _LOG2E).astype(jnp.bfloat16)

  logz = pl.pallas_call(
      functools.partial(_stats_body, nv),
      grid=(nv,),
      in_specs=[
          pl.BlockSpec((batch, emb), lambda v: (0, 0)),
          pl.BlockSpec((_TV, emb), lambda v: (v, 0)),
          pl.BlockSpec((1, _TV), lambda v: (0, v)),
      ],
      out_specs=pl.BlockSpec((batch, 1), lambda v: (0, 0)),
      out_shape=jax.ShapeDtypeStruct((batch, 1), jnp.float32),
      scratch_shapes=[
          pltpu.VMEM((batch, 1), jnp.float32),
          pltpu.VMEM((batch, 1), jnp.float32),
      ],
      compiler_params=pltpu.CompilerParams(
          dimension_semantics=("arbitrary",)),
  )(emb16, w16, b2s)

  out = pl.pallas_call(
      _project_body,
      grid=(nv,),
      in_specs=[
          pl.BlockSpec((batch, emb), lambda v: (0, 0)),
          pl.BlockSpec((_TV, emb), lambda v: (v, 0)),
          pl.BlockSpec((1, _TV), lambda v: (0, v)),
          pl.BlockSpec((batch, 1), lambda v: (0, 0)),
      ],
      out_specs=pl.BlockSpec((batch, _TV), lambda v: (0, v)),
      out_shape=jax.ShapeDtypeStruct((batch, vocab), jnp.float32),
      compiler_params=pltpu.CompilerParams(
          dimension_semantics=("parallel",)),
  )(embeds, W, b.reshape(1, -1), logz)
  return out


def kernel(inputs, emb_table, W, b):
  ctx, batch = inputs.shape
  vocab, emb = emb_table.shape
  idx = inputs.T.reshape(32, -1, 128)
  embeds = _make_gather_sum(vocab, emb, batch, ctx)(idx, emb_table)
  return _log_softmax_linear(embeds, W, b)


# speedup vs baseline: 2.2508x; 2.2508x over previous
import functools

import jax
import jax.numpy as jnp
from jax import lax
from jax.experimental import pallas as pl
from jax.experimental.pallas import tpu as pltpu
from jax.experimental.pallas import tpu_sc as plsc


def _make_gather_sum(vocab_rows, emb, batch, ctx):
  info = plsc.get_sparse_core_info()
  nc, ns = info.num_cores, info.num_subcores
  nw = nc * ns
  bpw = batch // nw
  ipw = bpw * ctx
  chunk = 128
  nchunk = ipw // chunk
  assert ipw % chunk == 0

  mesh = plsc.VectorSubcoreMesh(core_axis_name="c", subcore_axis_name="s")

  @functools.partial(
      pl.kernel,
      mesh=mesh,
      out_type=jax.ShapeDtypeStruct((batch, emb), jnp.float32),
      compiler_params=pltpu.CompilerParams(use_tc_tiling_on_sc=False),
      scratch_types=[
          pltpu.VMEM((nchunk, chunk), jnp.int32),
          pltpu.VMEM((ipw, emb), jnp.float32),
          pltpu.VMEM((bpw, emb), jnp.float32),
          pltpu.SemaphoreType.DMA,
      ],
  )
  def gather_sum(idx_hbm, table_hbm, out_hbm, idx_v, rows_v, acc_v, sem):
    wid = lax.axis_index("s") * nc + lax.axis_index("c")
    pltpu.sync_copy(idx_hbm.at[wid], idx_v)
    copies = []
    for j in range(nchunk):
      copies.append(
          pltpu.async_copy(
              table_hbm.at[idx_v.at[j]],
              rows_v.at[pl.ds(j * chunk, chunk)],
              sem,
          ))
    for c in copies:
      c.wait()

    def body(i, carry):
      for jj in range(emb // 16):
        sl = pl.ds(jj * 16, 16)
        acc = rows_v[i * ctx, sl]
        for c in range(1, ctx):
          acc = acc + rows_v[i * ctx + c, sl]
        acc_v[i, sl] = acc
      return carry

    lax.fori_loop(0, bpw, body, 0)
    pltpu.sync_copy(acc_v, out_hbm.at[pl.ds(wid * bpw, bpw)])

  return gather_sum


_TV = 1024
_LOG2E = 1.4426950408889634


def _stats_body(nv, emb_ref, w_ref, logz_ref, m_ref, s_ref):
  v = pl.program_id(0)

  @pl.when(v == 0)
  def _():
    m_ref[...] = jnp.full_like(m_ref, -jnp.inf)
    s_ref[...] = jnp.zeros_like(s_ref)

  logits2 = lax.dot_general(
      w_ref[...], emb_ref[...],
      dimension_numbers=(((0,), (0,)), ((), ())),
      preferred_element_type=jnp.float32,
  )
  tile_max = jnp.max(logits2, axis=0, keepdims=True)
  m_old = m_ref[...]
  m_new = jnp.maximum(m_old, tile_max)
  s_ref[...] = (s_ref[...] * jnp.exp2(m_old - m_new)
                + jnp.sum(jnp.exp2(logits2 - m_new), axis=0, keepdims=True))
  m_ref[...] = m_new

  @pl.when(v == nv - 1)
  def _():
    logz_ref[...] = (m_ref[...] + jnp.log2(s_ref[...])) * (1.0 / _LOG2E)


def _project_body(emb_ref, w_ref, logz_ref, out_ref):
  acc = lax.dot_general(
      w_ref[...], emb_ref[...],
      dimension_numbers=(((0,), (0,)), ((), ())),
      preferred_element_type=jnp.float32,
  )
  out_ref[...] = acc - logz_ref[...]


def _log_softmax_linear(embeds, W, b):
  batch, emb = embeds.shape
  vocab = W.shape[0]
  nv = pl.cdiv(vocab, _TV)
  vpad = nv * _TV - vocab

  wt_aug = jnp.concatenate(
      [jnp.pad(W.T, ((0, 0), (0, vpad))),
       jnp.pad(b, (0, vpad), constant_values=-1e30).reshape(1, -1)],
      axis=0)
  ones_row = jnp.ones((1, batch), jnp.float32)
  emb_aug = jnp.concatenate([embeds.T, ones_row], axis=0)
  emb_aug_s = emb_aug * _LOG2E

  logz = pl.pallas_call(
      functools.partial(_stats_body, nv),
      grid=(nv,),
      in_specs=[
          pl.BlockSpec((emb + 1, batch), lambda v: (0, 0)),
          pl.BlockSpec((emb + 1, _TV), lambda v: (0, v)),
      ],
      out_specs=pl.BlockSpec((1, batch), lambda v: (0, 0)),
      out_shape=jax.ShapeDtypeStruct((1, batch), jnp.float32),
      scratch_shapes=[
          pltpu.VMEM((1, batch), jnp.float32),
          pltpu.VMEM((1, batch), jnp.float32),
      ],
      compiler_params=pltpu.CompilerParams(
          dimension_semantics=("arbitrary",)),
  )(emb_aug_s, wt_aug)

  out_t = pl.pallas_call(
      _project_body,
      grid=(nv,),
      in_specs=[
          pl.BlockSpec((emb + 1, batch), lambda v: (0, 0)),
          pl.BlockSpec((emb + 1, _TV), lambda v: (0, v)),
          pl.BlockSpec((1, batch), lambda v: (0, 0)),
      ],
      out_specs=pl.BlockSpec((_TV, batch), lambda v: (v, 0)),
      out_shape=jax.ShapeDtypeStruct((vocab, batch), jnp.float32),
      compiler_params=pltpu.CompilerParams(
          dimension_semantics=("parallel",)),
  )(emb_aug, wt_aug, logz)
  return out_t.T


def kernel(inputs, emb_table, W, b):
  ctx, batch = inputs.shape
  vocab, emb = emb_table.shape
  idx = inputs.T.reshape(32, -1, 128)
  embeds = _make_gather_sum(vocab, emb, batch, ctx)(idx, emb_table)
  return _log_softmax_linear(embeds, W, b)


# speedup vs baseline: 2.3699x; 1.0529x over previous
import functools

import jax
import jax.numpy as jnp
from jax import lax
from jax.experimental import pallas as pl
from jax.experimental.pallas import tpu as pltpu
from jax.experimental.pallas import tpu_sc as plsc


def _make_gather_sum(vocab_rows, emb, batch, ctx):
  info = plsc.get_sparse_core_info()
  nc, ns = info.num_cores, info.num_subcores
  nw = nc * ns
  bpw = batch // nw
  ipw = bpw * ctx
  chunk = 128
  nchunk = ipw // chunk
  assert ipw % chunk == 0

  mesh = plsc.VectorSubcoreMesh(core_axis_name="c", subcore_axis_name="s")

  @functools.partial(
      pl.kernel,
      mesh=mesh,
      out_type=jax.ShapeDtypeStruct((batch, emb), jnp.float32),
      compiler_params=pltpu.CompilerParams(use_tc_tiling_on_sc=False),
      scratch_types=[
          pltpu.VMEM((nchunk, chunk), jnp.int32),
          pltpu.VMEM((ipw, emb), jnp.float32),
          pltpu.VMEM((bpw, emb), jnp.float32),
          pltpu.SemaphoreType.DMA,
      ],
  )
  def gather_sum(idx_hbm, table_hbm, out_hbm, idx_v, rows_v, acc_v, sem):
    wid = lax.axis_index("s") * nc + lax.axis_index("c")
    pltpu.sync_copy(idx_hbm.at[wid], idx_v)
    copies = []
    for j in range(nchunk):
      copies.append(
          pltpu.async_copy(
              table_hbm.at[idx_v.at[j]],
              rows_v.at[pl.ds(j * chunk, chunk)],
              sem,
          ))
    for c in copies:
      c.wait()

    def body(i, carry):
      for jj in range(emb // 16):
        sl = pl.ds(jj * 16, 16)
        acc = rows_v[i * ctx, sl]
        for c in range(1, ctx):
          acc = acc + rows_v[i * ctx + c, sl]
        acc_v[i, sl] = acc
      return carry

    lax.fori_loop(0, bpw, body, 0)
    pltpu.sync_copy(acc_v, out_hbm.at[pl.ds(wid * bpw, bpw)])

  return gather_sum


_TV_STATS = 2048
_TV_PROJ = 2048
_LOG2E = 1.4426950408889634


def _stats_body(nv, emb_ref, w_ref, logz_ref, m_ref, s_ref):
  v = pl.program_id(0)

  @pl.when(v == 0)
  def _():
    m_ref[...] = jnp.full_like(m_ref, -jnp.inf)
    s_ref[...] = jnp.zeros_like(s_ref)

  logits2 = lax.dot_general(
      w_ref[...], emb_ref[...],
      dimension_numbers=(((0,), (0,)), ((), ())),
      preferred_element_type=jnp.float32,
  )
  tile_max = jnp.max(logits2, axis=0, keepdims=True)
  m_old = m_ref[...]
  m_new = jnp.maximum(m_old, tile_max)
  s_ref[...] = (s_ref[...] * jnp.exp2(m_old - m_new)
                + jnp.sum(jnp.exp2(logits2 - m_new), axis=0, keepdims=True))
  m_ref[...] = m_new

  @pl.when(v == nv - 1)
  def _():
    logz_ref[...] = (m_ref[...] + jnp.log2(s_ref[...])) * (1.0 / _LOG2E)


def _project_body(emb_ref, w_ref, logz_ref, out_ref):
  acc = lax.dot_general(
      w_ref[...], emb_ref[...],
      dimension_numbers=(((0,), (0,)), ((), ())),
      preferred_element_type=jnp.float32,
  )
  out_ref[...] = acc - logz_ref[...]


def _log_softmax_linear(embeds, W, b):
  batch, emb = embeds.shape
  vocab = W.shape[0]
  nvs = pl.cdiv(vocab, _TV_STATS)
  nvp = pl.cdiv(vocab, _TV_PROJ)
  vtot = max(nvs * _TV_STATS, nvp * _TV_PROJ)
  vpad = vtot - vocab

  wt_aug = jnp.concatenate(
      [jnp.pad(W.T, ((0, 0), (0, vpad))),
       jnp.pad(b, (0, vpad), constant_values=-1e30).reshape(1, -1)],
      axis=0)
  ones_row = jnp.ones((1, batch), jnp.float32)
  emb_aug = jnp.concatenate([embeds.T, ones_row], axis=0)
  emb_aug_s = emb_aug * _LOG2E

  logz = pl.pallas_call(
      functools.partial(_stats_body, nvs),
      grid=(nvs,),
      in_specs=[
          pl.BlockSpec((emb + 1, batch), lambda v: (0, 0)),
          pl.BlockSpec((emb + 1, _TV_STATS), lambda v: (0, v)),
      ],
      out_specs=pl.BlockSpec((1, batch), lambda v: (0, 0)),
      out_shape=jax.ShapeDtypeStruct((1, batch), jnp.float32),
      scratch_shapes=[
          pltpu.VMEM((1, batch), jnp.float32),
          pltpu.VMEM((1, batch), jnp.float32),
      ],
      compiler_params=pltpu.CompilerParams(
          dimension_semantics=("arbitrary",)),
  )(emb_aug_s, wt_aug)

  out_t = pl.pallas_call(
      _project_body,
      grid=(nvp,),
      in_specs=[
          pl.BlockSpec((emb + 1, batch), lambda v: (0, 0)),
          pl.BlockSpec((emb + 1, _TV_PROJ), lambda v: (0, v)),
          pl.BlockSpec((1, batch), lambda v: (0, 0)),
      ],
      out_specs=pl.BlockSpec((_TV_PROJ, batch), lambda v: (v, 0)),
      out_shape=jax.ShapeDtypeStruct((vocab, batch), jnp.float32),
      compiler_params=pltpu.CompilerParams(
          dimension_semantics=("parallel",)),
  )(emb_aug, wt_aug, logz)
  return out_t.T


def kernel(inputs, emb_table, W, b):
  ctx, batch = inputs.shape
  vocab, emb = emb_table.shape
  idx = inputs.T.reshape(32, -1, 128)
  embeds = _make_gather_sum(vocab, emb, batch, ctx)(idx, emb_table)
  return _log_softmax_linear(embeds, W, b)


# speedup vs baseline: 2.4218x; 1.0219x over previous
import functools

import jax
import jax.numpy as jnp
from jax import lax
from jax.experimental import pallas as pl
from jax.experimental.pallas import tpu as pltpu
from jax.experimental.pallas import tpu_sc as plsc


def _make_gather_sum(vocab_rows, emb, batch, ctx):
  info = plsc.get_sparse_core_info()
  nc, ns = info.num_cores, info.num_subcores
  nw = nc * ns
  bpw = batch // nw
  ipw = bpw * ctx
  chunk = 128
  nchunk = ipw // chunk
  assert ipw % chunk == 0

  mesh = plsc.VectorSubcoreMesh(core_axis_name="c", subcore_axis_name="s")

  @functools.partial(
      pl.kernel,
      mesh=mesh,
      out_type=jax.ShapeDtypeStruct((batch, emb), jnp.float32),
      compiler_params=pltpu.CompilerParams(use_tc_tiling_on_sc=False),
      scratch_types=[
          pltpu.VMEM((nchunk, chunk), jnp.int32),
          pltpu.VMEM((ipw, emb), jnp.float32),
          pltpu.VMEM((bpw, emb), jnp.float32),
          pltpu.SemaphoreType.DMA,
      ],
  )
  def gather_sum(idx_hbm, table_hbm, out_hbm, idx_v, rows_v, acc_v, sem):
    wid = lax.axis_index("s") * nc + lax.axis_index("c")
    pltpu.sync_copy(idx_hbm.at[wid], idx_v)
    copies = []
    for j in range(nchunk):
      copies.append(
          pltpu.async_copy(
              table_hbm.at[idx_v.at[j]],
              rows_v.at[pl.ds(j * chunk, chunk)],
              sem,
          ))
    for c in copies:
      c.wait()

    def body(i, carry):
      for jj in range(emb // 16):
        sl = pl.ds(jj * 16, 16)
        acc = rows_v[i * ctx, sl]
        for c in range(1, ctx):
          acc = acc + rows_v[i * ctx + c, sl]
        acc_v[i, sl] = acc
      return carry

    lax.fori_loop(0, bpw, body, 0)
    pltpu.sync_copy(acc_v, out_hbm.at[pl.ds(wid * bpw, bpw)])

  return gather_sum


_TV_STATS = 2048
_TV_PROJ = 2048
_LOG2E = 1.4426950408889634


def _stats_body(nv, emb_ref, w_ref, logz_ref, m_ref, s_ref):
  v = pl.program_id(0)

  @pl.when(v == 0)
  def _():
    m_ref[...] = jnp.full_like(m_ref, -jnp.inf)
    s_ref[...] = jnp.zeros_like(s_ref)

  logits2 = lax.dot_general(
      w_ref[...], emb_ref[...],
      dimension_numbers=(((0,), (0,)), ((), ())),
      preferred_element_type=jnp.float32,
  ).astype(jnp.bfloat16)
  tile_max = jnp.max(logits2, axis=0, keepdims=True).astype(jnp.float32)
  m_old = m_ref[...]
  m_new = jnp.maximum(m_old, tile_max)
  e = jnp.exp2(logits2 - m_new.astype(jnp.bfloat16))
  ones_row = jnp.ones((1, e.shape[0]), jnp.bfloat16)
  s_tile = lax.dot_general(
      ones_row, e,
      dimension_numbers=(((1,), (0,)), ((), ())),
      preferred_element_type=jnp.float32,
  )
  s_ref[...] = s_ref[...] * jnp.exp2(m_old - m_new) + s_tile
  m_ref[...] = m_new

  @pl.when(v == nv - 1)
  def _():
    logz_ref[...] = (m_ref[...] + jnp.log2(s_ref[...])) * (1.0 / _LOG2E)


def _project_body(emb_ref, w_ref, logz_ref, out_ref):
  acc = lax.dot_general(
      w_ref[...], emb_ref[...],
      dimension_numbers=(((0,), (0,)), ((), ())),
      preferred_element_type=jnp.float32,
  )
  out_ref[...] = acc - logz_ref[...]


def _log_softmax_linear(embeds, W, b):
  batch, emb = embeds.shape
  vocab = W.shape[0]
  nvs = pl.cdiv(vocab, _TV_STATS)
  nvp = pl.cdiv(vocab, _TV_PROJ)
  vtot = max(nvs * _TV_STATS, nvp * _TV_PROJ)
  vpad = vtot - vocab

  wt_aug = jnp.concatenate(
      [jnp.pad(W.T, ((0, 0), (0, vpad))),
       jnp.pad(b, (0, vpad), constant_values=-1e30).reshape(1, -1)],
      axis=0)
  ones_row = jnp.ones((1, batch), jnp.float32)
  emb_aug = jnp.concatenate([embeds.T, ones_row], axis=0)
  emb_aug_s = (emb_aug * _LOG2E).astype(jnp.bfloat16)
  wt_aug_bf = wt_aug.astype(jnp.bfloat16)

  logz = pl.pallas_call(
      functools.partial(_stats_body, nvs),
      grid=(nvs,),
      in_specs=[
          pl.BlockSpec((emb + 1, batch), lambda v: (0, 0)),
          pl.BlockSpec((emb + 1, _TV_STATS), lambda v: (0, v)),
      ],
      out_specs=pl.BlockSpec((1, batch), lambda v: (0, 0)),
      out_shape=jax.ShapeDtypeStruct((1, batch), jnp.float32),
      scratch_shapes=[
          pltpu.VMEM((1, batch), jnp.float32),
          pltpu.VMEM((1, batch), jnp.float32),
      ],
      compiler_params=pltpu.CompilerParams(
          dimension_semantics=("arbitrary",)),
  )(emb_aug_s, wt_aug_bf)

  out_t = pl.pallas_call(
      _project_body,
      grid=(nvp,),
      in_specs=[
          pl.BlockSpec((emb + 1, batch), lambda v: (0, 0)),
          pl.BlockSpec((emb + 1, _TV_PROJ), lambda v: (0, v)),
          pl.BlockSpec((1, batch), lambda v: (0, 0)),
      ],
      out_specs=pl.BlockSpec((_TV_PROJ, batch), lambda v: (v, 0)),
      out_shape=jax.ShapeDtypeStruct((vocab, batch), jnp.float32),
      compiler_params=pltpu.CompilerParams(
          dimension_semantics=("parallel",)),
  )(emb_aug, wt_aug, logz)
  return out_t.T


def kernel(inputs, emb_table, W, b):
  ctx, batch = inputs.shape
  vocab, emb = emb_table.shape
  idx = inputs.T.reshape(32, -1, 128)
  embeds = _make_gather_sum(vocab, emb, batch, ctx)(idx, emb_table)
  return _log_softmax_linear(embeds, W, b)


# speedup vs baseline: 2.4663x; 1.0184x over previous
import functools

import jax
import jax.numpy as jnp
from jax import lax
from jax.experimental import pallas as pl
from jax.experimental.pallas import tpu as pltpu
from jax.experimental.pallas import tpu_sc as plsc


def _make_gather_sum(vocab_rows, emb, batch, ctx):
  info = plsc.get_sparse_core_info()
  nc, ns = info.num_cores, info.num_subcores
  nw = nc * ns
  bpw = batch // nw
  ipw = bpw * ctx
  chunk = 128
  nchunk = ipw // chunk
  assert ipw % chunk == 0

  mesh = plsc.VectorSubcoreMesh(core_axis_name="c", subcore_axis_name="s")

  @functools.partial(
      pl.kernel,
      mesh=mesh,
      out_type=jax.ShapeDtypeStruct((batch, emb), jnp.float32),
      compiler_params=pltpu.CompilerParams(use_tc_tiling_on_sc=False),
      scratch_types=[
          pltpu.VMEM((nchunk, chunk), jnp.int32),
          pltpu.VMEM((ipw, emb), jnp.float32),
          pltpu.VMEM((bpw, emb), jnp.float32),
          pltpu.SemaphoreType.DMA,
      ],
  )
  def gather_sum(idx_hbm, table_hbm, out_hbm, idx_v, rows_v, acc_v, sem):
    wid = lax.axis_index("s") * nc + lax.axis_index("c")
    pltpu.sync_copy(idx_hbm.at[wid], idx_v)
    copies = []
    for j in range(nchunk):
      copies.append(
          pltpu.async_copy(
              table_hbm.at[idx_v.at[j]],
              rows_v.at[pl.ds(j * chunk, chunk)],
              sem,
          ))
    for c in copies:
      c.wait()

    def body(i, carry):
      for jj in range(emb // 16):
        sl = pl.ds(jj * 16, 16)
        acc = rows_v[i * ctx, sl]
        for c in range(1, ctx):
          acc = acc + rows_v[i * ctx + c, sl]
        acc_v[i, sl] = acc
      return carry

    lax.fori_loop(0, bpw, body, 0)
    pltpu.sync_copy(acc_v, out_hbm.at[pl.ds(wid * bpw, bpw)])

  return gather_sum


_TV_STATS = 2048
_TV_PROJ = 2048
_LOG2E = 1.4426950408889634


def _fused_body(nvs, embs_ref, emb_ref, w_ref, out_ref, m_ref, s_ref,
                logz_ref):
  v = pl.program_id(0)

  @pl.when(v == 0)
  def _():
    m_ref[...] = jnp.full_like(m_ref, -jnp.inf)
    s_ref[...] = jnp.zeros_like(s_ref)

  @pl.when(v < nvs)
  def _():
    logits2 = lax.dot_general(
        w_ref[...], embs_ref[...],
        dimension_numbers=(((0,), (0,)), ((), ())),
        preferred_element_type=jnp.float32,
    )
    tile_max = jnp.max(logits2, axis=0, keepdims=True)
    m_old = m_ref[...]
    m_new = jnp.maximum(m_old, tile_max)
    s_ref[...] = (s_ref[...] * jnp.exp2(m_old - m_new)
                  + jnp.sum(jnp.exp2(logits2 - m_new), axis=0, keepdims=True))
    m_ref[...] = m_new

  @pl.when(v == nvs - 1)
  def _():
    logz_ref[...] = (m_ref[...] + jnp.log2(s_ref[...])) * (1.0 / _LOG2E)

  @pl.when(v >= nvs)
  def _():
    acc = lax.dot_general(
        w_ref[...], emb_ref[...],
        dimension_numbers=(((0,), (0,)), ((), ())),
        preferred_element_type=jnp.float32,
    )
    out_ref[...] = acc - logz_ref[...]


def _log_softmax_linear(embeds, W, b):
  batch, emb = embeds.shape
  vocab = W.shape[0]
  nvs = pl.cdiv(vocab, _TV_STATS)
  nvp = pl.cdiv(vocab, _TV_PROJ)
  vtot = max(nvs * _TV_STATS, nvp * _TV_PROJ)
  vpad = vtot - vocab

  wt_aug = jnp.concatenate(
      [jnp.pad(W.T, ((0, 0), (0, vpad))),
       jnp.pad(b, (0, vpad), constant_values=-1e30).reshape(1, -1)],
      axis=0)
  ones_row = jnp.ones((1, batch), jnp.float32)
  emb_aug = jnp.concatenate([embeds.T, ones_row], axis=0)
  emb_aug_s = emb_aug * _LOG2E

  out_t = pl.pallas_call(
      functools.partial(_fused_body, nvs),
      grid=(nvs + nvp,),
      in_specs=[
          pl.BlockSpec((emb + 1, batch), lambda v: (0, 0)),
          pl.BlockSpec((emb + 1, batch), lambda v: (0, 0)),
          pl.BlockSpec((emb + 1, _TV_STATS),
                       lambda v: (0, jnp.where(v < nvs, v, v - nvs))),
      ],
      out_specs=pl.BlockSpec((_TV_PROJ, batch),
                             lambda v: (jnp.where(v < nvs, 0, v - nvs), 0)),
      out_shape=jax.ShapeDtypeStruct((vocab, batch), jnp.float32),
      scratch_shapes=[
          pltpu.VMEM((1, batch), jnp.float32),
          pltpu.VMEM((1, batch), jnp.float32),
          pltpu.VMEM((1, batch), jnp.float32),
      ],
      compiler_params=pltpu.CompilerParams(
          dimension_semantics=("arbitrary",)),
  )(emb_aug_s, emb_aug, wt_aug)
  return out_t.T


def kernel(inputs, emb_table, W, b):
  ctx, batch = inputs.shape
  vocab, emb = emb_table.shape
  idx = inputs.T.reshape(32, -1, 128)
  embeds = _make_gather_sum(vocab, emb, batch, ctx)(idx, emb_table)
  return _log_softmax_linear(embeds, W, b)
